# baseline (speedup 1.0000x reference)
import functools

import jax
import jax.numpy as jnp
from jax import lax
from jax.experimental import pallas as pl
from jax.experimental.pallas import tpu as pltpu
from jax.experimental.pallas import tpu_sc as plsc

B, U, W, V, D = 256, 20, 64, 30000, 128
NC, NS = 2, 16
NW = NC * NS
DPW = B // NW
ROWS = DPW * U
CG = D // 16


@functools.cache
def _make_sc_pool():
  mesh = plsc.VectorSubcoreMesh(core_axis_name="c", subcore_axis_name="s")

  @functools.partial(
      pl.kernel,
      mesh=mesh,
      out_type=jax.ShapeDtypeStruct((B * U, D), jnp.float32),
      scratch_types=[
          pltpu.VMEM((ROWS * W,), jnp.int32),
          pltpu.VMEM((ROWS + 16,), jnp.int32),
          pltpu.VMEM((W, D), jnp.float32),
          pltpu.VMEM((W, D), jnp.float32),
          pltpu.VMEM((U, DPW, D), jnp.float32),
          pltpu.SemaphoreType.DMA,
          pltpu.SemaphoreType.DMA,
      ],
  )
  def sc_pool(tok_hbm, eff_hbm, emb_hbm, out_hbm,
              idx_v, eff_v, buf0, buf1, out_v, sem0, sem1):
    bufs = (buf0, buf1)
    sems = (sem0, sem1)
    wid = lax.axis_index("s") * NC + lax.axis_index("c")
    base = wid * ROWS
    pltpu.sync_copy(tok_hbm.at[pl.ds(base * W, ROWS * W)], idx_v)
    pltpu.sync_copy(eff_hbm.at[pl.ds(base, ROWS)], eff_v.at[pl.ds(0, ROWS)])

    def eff_at(j):
      return eff_v[pl.ds(j, 16)][0]

    def copy_for(j, nrows, buf, sem):
      return pltpu.make_async_copy(
          emb_hbm.at[idx_v.at[pl.ds(j * W, nrows)]],
          buf.at[pl.ds(0, nrows)], sem)

    def issue(j, buf, sem):
      nh = (eff_at(j) + 31) >> 5
      for g in range(1, W // 32 + 1):
        @pl.when(nh == g)
        def _():
          copy_for(j, 32 * g, buf, sem).start()

    def wait_acc(j, buf, sem):
      eff = eff_at(j)
      nh = (eff + 31) >> 5
      for g in range(1, W // 32 + 1):
        @pl.when(nh == g)
        def _():
          copy_for(j, 32 * g, buf, sem).wait()
      nb = (eff + 15) >> 4
      def blk(i, acc):
        accs = list(acc)
        for rr in range(16):
          r = i * 16 + rr
          wgt = (r < eff).astype(jnp.float32)
          for c in range(CG):
            accs[c] = accs[c] + wgt * buf[r, pl.ds(c * 16, 16)]
        return tuple(accs)

      acc0 = tuple(jnp.zeros((16,), jnp.float32) for _ in range(CG))
      acc = lax.fori_loop(0, nb, blk, acc0)
      b_local = j // U
      u = j - b_local * U
      for c in range(CG):
        out_v[u, b_local, pl.ds(c * 16, 16)] = acc[c]

    def body(j, carry):
      issue(j, bufs[0], sems[0])
      wait_acc(j, bufs[0], sems[0])
      return carry

    lax.fori_loop(0, ROWS, body, 0)
    for u in range(U):
      pltpu.sync_copy(out_v.at[u], out_hbm.at[pl.ds(u * B + wid * DPW, DPW)])

  return sc_pool


def _tc_body(x_ref, eff_ref, clen_ref, wu_ref, bu_ref, wd_ref, bd_ref, o_ref):
  eff = eff_ref[...]
  x = x_ref[...] / jnp.maximum(eff, 1).astype(jnp.float32)
  y = jnp.tanh(jnp.dot(x, wu_ref[...], preferred_element_type=jnp.float32)
               + bu_ref[...])
  y = y * (eff > 0).astype(jnp.float32)
  clen = clen_ref[...]
  inv = 1.0 / jnp.maximum(clen, 1).astype(jnp.float32)
  acc = jnp.zeros((B, D), jnp.float32)
  for u in range(U):
    wcol = jnp.where(u < clen, inv, 0.0)
    acc = acc + wcol * y[u * B:(u + 1) * B, :]
  c = jnp.tanh(jnp.dot(acc, wd_ref[...], preferred_element_type=jnp.float32)
               + bd_ref[...])
  o_ref[...] = c * (clen > 0).astype(jnp.float32)


_tc_finish = pl.pallas_call(
    _tc_body,
    out_shape=jax.ShapeDtypeStruct((B, D), jnp.float32),
)


def kernel(contexts, context_utterance_lengths, context_lengths,
           emb_table, W_u, b_u, W_d, b_d):
  contexts = contexts.astype(jnp.int32)
  cul = context_utterance_lengths.astype(jnp.int32)
  clen = context_lengths.astype(jnp.int32)
  upos = jnp.arange(U, dtype=jnp.int32)[None, :]
  eff = jnp.where(upos < clen[:, None], jnp.clip(cul, 0, W), 0)
  tok = contexts.reshape(-1)
  sums = _make_sc_pool()(tok, eff.reshape(-1), emb_table.astype(jnp.float32))
  eff_t = eff.T.reshape(-1, 1)
  ctx = _tc_finish(sums, eff_t, clen[:, None],
                   W_u.astype(jnp.float32), b_u.astype(jnp.float32)[None, :],
                   W_d.astype(jnp.float32), b_d.astype(jnp.float32)[None, :])
  return ctx


# speedup vs baseline: 2.0232x; 2.0232x over previous
import functools

import jax
import jax.numpy as jnp
from jax import lax
from jax.experimental import pallas as pl
from jax.experimental.pallas import tpu as pltpu
from jax.experimental.pallas import tpu_sc as plsc

B, U, W, V, D = 256, 20, 64, 30000, 128
NC, NS = 2, 16
NW = NC * NS
DPW = B // NW
ROWS = DPW * U
CG = D // 16


@functools.cache
def _make_sc_pool():
  mesh = plsc.VectorSubcoreMesh(core_axis_name="c", subcore_axis_name="s")

  @functools.partial(
      pl.kernel,
      mesh=mesh,
      out_type=jax.ShapeDtypeStruct((B * U, D), jnp.float32),
      scratch_types=[
          pltpu.VMEM((ROWS * W,), jnp.int32),
          pltpu.VMEM((ROWS + 16,), jnp.int32),
          pltpu.VMEM((4, W, D), jnp.float32),
          pltpu.VMEM((U, DPW, D), jnp.float32),
          pltpu.SemaphoreType.DMA((4,)),
      ],
  )
  def sc_pool(tok_hbm, eff_hbm, emb_hbm, out_hbm,
              idx_v, eff_v, buf, out_v, sem):
    wid = lax.axis_index("s") * NC + lax.axis_index("c")
    base = wid * ROWS
    pltpu.sync_copy(tok_hbm.at[pl.ds(base * W, ROWS * W)], idx_v)
    pltpu.sync_copy(eff_hbm.at[pl.ds(base, ROWS)], eff_v.at[pl.ds(0, ROWS)])

    def eff_at(j):
      return eff_v[pl.ds(j, 16)][0]

    def copy_for(j, nrows, s):
      return pltpu.make_async_copy(
          emb_hbm.at[idx_v.at[pl.ds(j * W, nrows)]],
          buf.at[s, pl.ds(0, nrows)], sem.at[s])

    def issue(j, s):
      nb = (eff_at(j) + 15) >> 4
      for g in range(1, W // 16 + 1):
        @pl.when(nb == g)
        def _():
          copy_for(j, 16 * g, s).start()

    def wait_acc(j, s):
      eff = eff_at(j)
      nb = (eff + 15) >> 4
      for g in range(1, W // 16 + 1):
        @pl.when(nb == g)
        def _():
          copy_for(j, 16 * g, s).wait()
      def blk(i, acc):
        accs = list(acc)
        for rr in range(16):
          r = i * 16 + rr
          wgt = (r < eff).astype(jnp.float32)
          for c in range(CG):
            accs[c] = accs[c] + wgt * buf[s, r, pl.ds(c * 16, 16)]
        return tuple(accs)

      acc0 = tuple(jnp.zeros((16,), jnp.float32) for _ in range(CG))
      acc = lax.fori_loop(0, nb, blk, acc0)
      b_local = j // U
      u = j - b_local * U
      for c in range(CG):
        out_v[u, b_local, pl.ds(c * 16, 16)] = acc[c]

    for k in range(3):
      issue(k, k)

    def body(j, carry):
      @pl.when(j + 3 < ROWS)
      def _():
        issue(j + 3, lax.rem(j + 3, 4))

      wait_acc(j, lax.rem(j, 4))
      return carry

    lax.fori_loop(0, ROWS, body, 0)
    for u in range(U):
      pltpu.sync_copy(out_v.at[u], out_hbm.at[pl.ds(u * B + wid * DPW, DPW)])

  return sc_pool


def _tc_body(x_ref, eff_ref, clen_ref, wu_ref, bu_ref, wd_ref, bd_ref, o_ref):
  eff = eff_ref[...]
  x = x_ref[...] / jnp.maximum(eff, 1).astype(jnp.float32)
  y = jnp.tanh(jnp.dot(x, wu_ref[...], preferred_element_type=jnp.float32)
               + bu_ref[...])
  y = y * (eff > 0).astype(jnp.float32)
  clen = clen_ref[...]
  inv = 1.0 / jnp.maximum(clen, 1).astype(jnp.float32)
  acc = jnp.zeros((B, D), jnp.float32)
  for u in range(U):
    wcol = jnp.where(u < clen, inv, 0.0)
    acc = acc + wcol * y[u * B:(u + 1) * B, :]
  c = jnp.tanh(jnp.dot(acc, wd_ref[...], preferred_element_type=jnp.float32)
               + bd_ref[...])
  o_ref[...] = c * (clen > 0).astype(jnp.float32)


_tc_finish = pl.pallas_call(
    _tc_body,
    out_shape=jax.ShapeDtypeStruct((B, D), jnp.float32),
)


def kernel(contexts, context_utterance_lengths, context_lengths,
           emb_table, W_u, b_u, W_d, b_d):
  contexts = contexts.astype(jnp.int32)
  cul = context_utterance_lengths.astype(jnp.int32)
  clen = context_lengths.astype(jnp.int32)
  upos = jnp.arange(U, dtype=jnp.int32)[None, :]
  eff = jnp.where(upos < clen[:, None], jnp.clip(cul, 0, W), 0)
  tok = contexts.reshape(-1)
  sums = _make_sc_pool()(tok, eff.reshape(-1), emb_table.astype(jnp.float32))
  eff_t = eff.T.reshape(-1, 1)
  ctx = _tc_finish(sums, eff_t, clen[:, None],
                   W_u.astype(jnp.float32), b_u.astype(jnp.float32)[None, :],
                   W_d.astype(jnp.float32), b_d.astype(jnp.float32)[None, :])
  return ctx


# speedup vs baseline: 2.1930x; 1.0839x over previous
import functools

import jax
import jax.numpy as jnp
from jax import lax
from jax.experimental import pallas as pl
from jax.experimental.pallas import tpu as pltpu
from jax.experimental.pallas import tpu_sc as plsc

B, U, W, V, D = 256, 20, 64, 30000, 128
NC, NS = 2, 16
NW = NC * NS
DPW = B // NW
ROWS = DPW * U
CG = D // 16


@functools.cache
def _make_sc_pool():
  mesh = plsc.VectorSubcoreMesh(core_axis_name="c", subcore_axis_name="s")

  @functools.partial(
      pl.kernel,
      mesh=mesh,
      out_type=jax.ShapeDtypeStruct((B * U, D), jnp.float32),
      scratch_types=[
          pltpu.VMEM((ROWS * W,), jnp.int32),
          pltpu.VMEM((ROWS + 16,), jnp.int32),
          pltpu.VMEM((8, W, D), jnp.float32),
          pltpu.VMEM((U, DPW, D), jnp.float32),
          pltpu.SemaphoreType.DMA((8,)),
      ],
  )
  def sc_pool(tok_hbm, eff_hbm, emb_hbm, out_hbm,
              idx_v, eff_v, buf, out_v, sem):
    wid = lax.axis_index("s") * NC + lax.axis_index("c")
    base = wid * ROWS
    pltpu.sync_copy(tok_hbm.at[pl.ds(base * W, ROWS * W)], idx_v)
    pltpu.sync_copy(eff_hbm.at[pl.ds(base, ROWS)], eff_v.at[pl.ds(0, ROWS)])

    def eff_at(j):
      return eff_v[pl.ds(j, 16)][0]

    def copy_for(j, nrows, s):
      return pltpu.make_async_copy(
          emb_hbm.at[idx_v.at[pl.ds(j * W, nrows)]],
          buf.at[s, pl.ds(0, nrows)], sem.at[s])

    def issue(j, s):
      nb = (eff_at(j) + 15) >> 4
      for g in range(1, W // 16 + 1):
        @pl.when(nb == g)
        def _():
          copy_for(j, 16 * g, s).start()

    def wait_acc(j, s):
      eff = eff_at(j)
      nb = (eff + 15) >> 4
      for g in range(1, W // 16 + 1):
        @pl.when(nb == g)
        def _():
          copy_for(j, 16 * g, s).wait()
      def blk(i, acc):
        accs = list(acc)
        for rr in range(16):
          r = i * 16 + rr
          wgt = (r < eff).astype(jnp.float32)
          for c in range(CG):
            accs[c] = accs[c] + wgt * buf[s, r, pl.ds(c * 16, 16)]
        return tuple(accs)

      acc0 = tuple(jnp.zeros((16,), jnp.float32) for _ in range(CG))
      acc = lax.fori_loop(0, nb, blk, acc0)
      b_local = j // U
      u = j - b_local * U
      for c in range(CG):
        out_v[u, b_local, pl.ds(c * 16, 16)] = acc[c]

    for k in range(7):
      issue(k, k)

    def body(j, carry):
      @pl.when(j + 7 < ROWS)
      def _():
        issue(j + 7, lax.rem(j + 7, 8))

      wait_acc(j, lax.rem(j, 8))
      return carry

    lax.fori_loop(0, ROWS, body, 0)
    for u in range(U):
      pltpu.sync_copy(out_v.at[u], out_hbm.at[pl.ds(u * B + wid * DPW, DPW)])

  return sc_pool


def _tc_body(x_ref, eff_ref, clen_ref, wu_ref, bu_ref, wd_ref, bd_ref, o_ref):
  eff = eff_ref[...]
  x = x_ref[...] / jnp.maximum(eff, 1).astype(jnp.float32)
  y = jnp.tanh(jnp.dot(x, wu_ref[...], preferred_element_type=jnp.float32)
               + bu_ref[...])
  y = y * (eff > 0).astype(jnp.float32)
  clen = clen_ref[...]
  inv = 1.0 / jnp.maximum(clen, 1).astype(jnp.float32)
  acc = jnp.zeros((B, D), jnp.float32)
  for u in range(U):
    wcol = jnp.where(u < clen, inv, 0.0)
    acc = acc + wcol * y[u * B:(u + 1) * B, :]
  c = jnp.tanh(jnp.dot(acc, wd_ref[...], preferred_element_type=jnp.float32)
               + bd_ref[...])
  o_ref[...] = c * (clen > 0).astype(jnp.float32)


_tc_finish = pl.pallas_call(
    _tc_body,
    out_shape=jax.ShapeDtypeStruct((B, D), jnp.float32),
)


def kernel(contexts, context_utterance_lengths, context_lengths,
           emb_table, W_u, b_u, W_d, b_d):
  contexts = contexts.astype(jnp.int32)
  cul = context_utterance_lengths.astype(jnp.int32)
  clen = context_lengths.astype(jnp.int32)
  upos = jnp.arange(U, dtype=jnp.int32)[None, :]
  eff = jnp.where(upos < clen[:, None], jnp.clip(cul, 0, W), 0)
  tok = contexts.reshape(-1)
  sums = _make_sc_pool()(tok, eff.reshape(-1), emb_table.astype(jnp.float32))
  eff_t = eff.T.reshape(-1, 1)
  ctx = _tc_finish(sums, eff_t, clen[:, None],
                   W_u.astype(jnp.float32), b_u.astype(jnp.float32)[None, :],
                   W_d.astype(jnp.float32), b_d.astype(jnp.float32)[None, :])
  return ctx
